# Initial kernel scaffold; baseline (speedup 1.0000x reference)
import jax
import jax.numpy as jnp
from jax.experimental import pallas as pl


def kernel(x, edge_index, lstm_params, W1, b1, W2, b2):
    raise NotImplementedError("write your pallas kernel here")


# speedup vs baseline: 10.7332x; 10.7332x over previous
import functools

import jax
import jax.numpy as jnp
from jax import lax
from jax.experimental import pallas as pl
from jax.experimental.pallas import tpu as pltpu
from jax.experimental.pallas import tpu_sc as plsc

NN = 10000
HID = 128
NE = 320000
NC, NS = 2, 16
NW = NC * NS
EPW = NE // NW
VL = 16
CHUNK = 1000
GRID = NN // CHUNK

_f32 = jnp.float32


def _proj0_body(x_ref, wf_ref, wb_ref, bf_ref, bb_ref, xf_ref, xb_ref):
    x = x_ref[...]
    xf_ref[...] = jnp.dot(x, wf_ref[...], preferred_element_type=_f32) + bf_ref[...]
    xb_ref[...] = jnp.dot(x, wb_ref[...], preferred_element_type=_f32) + bb_ref[...]


def _proj0(x, wfT, wbT, bf, bb):
    return pl.pallas_call(
        _proj0_body,
        grid=(GRID,),
        in_specs=[
            pl.BlockSpec((CHUNK, HID), lambda i: (i, 0)),
            pl.BlockSpec((HID, 4 * HID), lambda i: (0, 0)),
            pl.BlockSpec((HID, 4 * HID), lambda i: (0, 0)),
            pl.BlockSpec((1, 4 * HID), lambda i: (0, 0)),
            pl.BlockSpec((1, 4 * HID), lambda i: (0, 0)),
        ],
        out_specs=[
            pl.BlockSpec((CHUNK, 4 * HID), lambda i: (i, 0)),
            pl.BlockSpec((CHUNK, 4 * HID), lambda i: (i, 0)),
        ],
        out_shape=[
            jax.ShapeDtypeStruct((NN, 4 * HID), _f32),
            jax.ShapeDtypeStruct((NN, 4 * HID), _f32),
        ],
    )(x, wfT, wbT, bf, bb)


def _proj1_body(hf_ref, hb_ref, af_ref, bf_ref, ab_ref, bb_ref,
                cf_ref, cb_ref, xf_ref, xb_ref):
    hf = hf_ref[...]
    hb = hb_ref[...]
    xf_ref[...] = (jnp.dot(hf, af_ref[...], preferred_element_type=_f32)
                   + jnp.dot(hb, bf_ref[...], preferred_element_type=_f32)
                   + cf_ref[...])
    xb_ref[...] = (jnp.dot(hf, ab_ref[...], preferred_element_type=_f32)
                   + jnp.dot(hb, bb_ref[...], preferred_element_type=_f32)
                   + cb_ref[...])


def _proj1(hf, hb, afT, bfT, abT, bbT, cf, cb):
    return pl.pallas_call(
        _proj1_body,
        grid=(GRID,),
        in_specs=[
            pl.BlockSpec((CHUNK, HID), lambda i: (i, 0)),
            pl.BlockSpec((CHUNK, HID), lambda i: (i, 0)),
            pl.BlockSpec((HID, 4 * HID), lambda i: (0, 0)),
            pl.BlockSpec((HID, 4 * HID), lambda i: (0, 0)),
            pl.BlockSpec((HID, 4 * HID), lambda i: (0, 0)),
            pl.BlockSpec((HID, 4 * HID), lambda i: (0, 0)),
            pl.BlockSpec((1, 4 * HID), lambda i: (0, 0)),
            pl.BlockSpec((1, 4 * HID), lambda i: (0, 0)),
        ],
        out_specs=[
            pl.BlockSpec((CHUNK, 4 * HID), lambda i: (i, 0)),
            pl.BlockSpec((CHUNK, 4 * HID), lambda i: (i, 0)),
        ],
        out_shape=[
            jax.ShapeDtypeStruct((NN, 4 * HID), _f32),
            jax.ShapeDtypeStruct((NN, 4 * HID), _f32),
        ],
    )(hf, hb, afT, bfT, abT, bbT, cf, cb)


def _scan_body(xf_ref, xb_ref, wf_ref, wb_ref, hf_out, hb_out,
               hf_s, cf_s, hb_s, cb_s):
    @pl.when(pl.program_id(0) == 0)
    def _():
        hf_s[...] = jnp.zeros((1, HID), _f32)
        cf_s[...] = jnp.zeros((1, HID), _f32)
        hb_s[...] = jnp.zeros((1, HID), _f32)
        cb_s[...] = jnp.zeros((1, HID), _f32)

    wf = wf_ref[...]
    wb = wb_ref[...]

    def step(t, carry):
        hf, cf, hb, cb = carry
        gf = xf_ref[pl.ds(t, 1), :] + jnp.dot(hf, wf, preferred_element_type=_f32)
        i = jax.nn.sigmoid(gf[:, 0:HID])
        f = jax.nn.sigmoid(gf[:, HID:2 * HID])
        g = jnp.tanh(gf[:, 2 * HID:3 * HID])
        o = jax.nn.sigmoid(gf[:, 3 * HID:4 * HID])
        cf = f * cf + i * g
        hf = o * jnp.tanh(cf)
        hf_out[pl.ds(t, 1), :] = hf

        tb = CHUNK - 1 - t
        gb = xb_ref[pl.ds(tb, 1), :] + jnp.dot(hb, wb, preferred_element_type=_f32)
        ib = jax.nn.sigmoid(gb[:, 0:HID])
        fb = jax.nn.sigmoid(gb[:, HID:2 * HID])
        gg = jnp.tanh(gb[:, 2 * HID:3 * HID])
        ob = jax.nn.sigmoid(gb[:, 3 * HID:4 * HID])
        cb = fb * cb + ib * gg
        hb = ob * jnp.tanh(cb)
        hb_out[pl.ds(tb, 1), :] = hb
        return hf, cf, hb, cb

    carry = (hf_s[...], cf_s[...], hb_s[...], cb_s[...])
    hf, cf, hb, cb = lax.fori_loop(0, CHUNK, step, carry)
    hf_s[...] = hf
    cf_s[...] = cf
    hb_s[...] = hb
    cb_s[...] = cb


def _bilstm_scan(xf, xb, whfT, whbT):
    return pl.pallas_call(
        _scan_body,
        grid=(GRID,),
        in_specs=[
            pl.BlockSpec((CHUNK, 4 * HID), lambda i: (i, 0)),
            pl.BlockSpec((CHUNK, 4 * HID), lambda i: (GRID - 1 - i, 0)),
            pl.BlockSpec((HID, 4 * HID), lambda i: (0, 0)),
            pl.BlockSpec((HID, 4 * HID), lambda i: (0, 0)),
        ],
        out_specs=[
            pl.BlockSpec((CHUNK, HID), lambda i: (i, 0)),
            pl.BlockSpec((CHUNK, HID), lambda i: (GRID - 1 - i, 0)),
        ],
        out_shape=[
            jax.ShapeDtypeStruct((NN, HID), _f32),
            jax.ShapeDtypeStruct((NN, HID), _f32),
        ],
        scratch_shapes=[pltpu.VMEM((1, HID), _f32)] * 4,
    )(xf, xb, whfT, whbT)


def _pproj_body(hf_ref, hb_ref, w1a_ref, w1b_ref, w2p_ref, p_ref):
    wca = jnp.dot(w1a_ref[...], w2p_ref[...], preferred_element_type=_f32)
    wcb = jnp.dot(w1b_ref[...], w2p_ref[...], preferred_element_type=_f32)
    p_ref[...] = (jnp.dot(hf_ref[...], wca, preferred_element_type=_f32)
                  + jnp.dot(hb_ref[...], wcb, preferred_element_type=_f32))


def _pproj(hf, hb, w1a, w1b, w2p):
    return pl.pallas_call(
        _pproj_body,
        grid=(GRID,),
        in_specs=[
            pl.BlockSpec((CHUNK, HID), lambda i: (i, 0)),
            pl.BlockSpec((CHUNK, HID), lambda i: (i, 0)),
            pl.BlockSpec((HID, 4 * HID), lambda i: (0, 0)),
            pl.BlockSpec((HID, 4 * HID), lambda i: (0, 0)),
            pl.BlockSpec((4 * HID, HID), lambda i: (0, 0)),
        ],
        out_specs=pl.BlockSpec((CHUNK, HID), lambda i: (i, 0)),
        out_shape=jax.ShapeDtypeStruct((NN, HID), _f32),
    )(hf, hb, w1a, w1b, w2p)


def _sc_mesh():
    return plsc.VectorSubcoreMesh(core_axis_name="c", subcore_axis_name="s",
                                  num_cores=NC, num_subcores=NS)


def _zero_vmem(ref):
    def zstep(i, _):
        ref[pl.ds(i * VL, VL)] = jnp.zeros((VL,), _f32)
        return 0
    lax.fori_loop(0, NN // VL, zstep, 0)


def _deg_body(dst_hbm, out_hbm, idx_v, acc_v):
    wid = lax.axis_index("s") * NC + lax.axis_index("c")
    pltpu.sync_copy(dst_hbm.at[pl.ds(wid * EPW, EPW)], idx_v)
    _zero_vmem(acc_v)
    ones = jnp.ones((VL,), _f32)

    def step(i, _):
        d = idx_v[pl.ds(i * VL, VL)]
        plsc.addupdate_scatter(acc_v, [d], ones)
        return 0

    lax.fori_loop(0, EPW // VL, step, 0)
    pltpu.sync_copy(acc_v, out_hbm.at[wid])


def _sc_degree(dst):
    return pl.kernel(
        _deg_body,
        out_type=jax.ShapeDtypeStruct((NW, NN), _f32),
        mesh=_sc_mesh(),
        compiler_params=pltpu.CompilerParams(needs_layout_passes=False),
        scratch_types=[
            pltpu.VMEM((EPW,), jnp.int32),
            pltpu.VMEM((NN,), _f32),
        ],
    )(dst)


def _make_agg_body(nch):
    def body(src_hbm, dst_hbm, *rest):
        tabs_hbm = rest[:nch]
        dis_hbm = rest[nch]
        outs_hbm = rest[nch + 1:nch + 2 + nch]
        scr = rest[nch + 2 + nch:]
        src_v, dst_v = scr[0], scr[1]
        tab_v = scr[2:2 + nch]
        dis_v = scr[2 + nch]
        acc_v = scr[3 + nch:3 + nch + nch + 1]

        wid = lax.axis_index("s") * NC + lax.axis_index("c")
        pltpu.sync_copy(src_hbm.at[pl.ds(wid * EPW, EPW)], src_v)
        pltpu.sync_copy(dst_hbm.at[pl.ds(wid * EPW, EPW)], dst_v)
        for c in range(nch):
            pltpu.sync_copy(tabs_hbm[c], tab_v[c])
        pltpu.sync_copy(dis_hbm, dis_v)
        for a in acc_v:
            _zero_vmem(a)

        def step(i, _):
            s = src_v[pl.ds(i * VL, VL)]
            d = dst_v[pl.ds(i * VL, VL)]
            ds = plsc.load_gather(dis_v, [s])
            for c in range(nch):
                v = plsc.load_gather(tab_v[c], [s]) * ds
                plsc.addupdate_scatter(acc_v[c], [d], v)
            plsc.addupdate_scatter(acc_v[nch], [d], ds)
            return 0

        lax.fori_loop(0, EPW // VL, step, 0)
        for c in range(nch + 1):
            pltpu.sync_copy(acc_v[c], outs_hbm[c].at[wid])

    return body


def _sc_aggregate(src, dst, tabs, dis):
    nch = len(tabs)
    out_t = [jax.ShapeDtypeStruct((NW, NN), _f32)] * (nch + 1)
    scratch = ([pltpu.VMEM((EPW,), jnp.int32)] * 2
               + [pltpu.VMEM((NN,), _f32)] * (nch + 1)
               + [pltpu.VMEM((NN,), _f32)] * (nch + 1))
    return pl.kernel(
        _make_agg_body(nch),
        out_type=out_t,
        mesh=_sc_mesh(),
        compiler_params=pltpu.CompilerParams(needs_layout_passes=False),
        scratch_types=scratch,
    )(src, dst, *tabs, dis)


def _norm_body(degp_ref, dis_ref):
    deg = jnp.sum(degp_ref[...], axis=0, keepdims=True) + 1.0
    dis_ref[...] = lax.rsqrt(deg)


def _norm(degp):
    return pl.pallas_call(
        _norm_body,
        out_shape=jax.ShapeDtypeStruct((1, NN), _f32),
    )(degp)


def _mid_body(a0_ref, a1_ref, ar_ref, dis_ref, p0_ref, p1_ref,
              z0_ref, z1_ref, r_ref):
    d = dis_ref[...]
    d2 = d * d
    z0_ref[...] = jnp.sum(a0_ref[...], axis=0, keepdims=True) * d + p0_ref[...] * d2
    z1_ref[...] = jnp.sum(a1_ref[...], axis=0, keepdims=True) * d + p1_ref[...] * d2
    r_ref[...] = jnp.sum(ar_ref[...], axis=0, keepdims=True) * d + d2


def _mid(a0, a1, ar, dis, p0, p1):
    return pl.pallas_call(
        _mid_body,
        out_shape=[jax.ShapeDtypeStruct((1, NN), _f32)] * 3,
    )(a0, a1, ar, dis, p0, p1)


def _final_body(a0_ref, a1_ref, dis_ref, z0_ref, z1_ref, r_ref,
                b1_ref, w2p_ref, b2_ref, o0_ref, o1_ref):
    d = dis_ref[...]
    d2 = d * d
    bw = jnp.dot(b1_ref[...], w2p_ref[...], preferred_element_type=_f32)
    bw0 = bw[0:1, 0:1]
    bw1 = bw[0:1, 1:2]
    b20 = b2_ref[0:1, 0:1]
    b21 = b2_ref[0:1, 1:2]
    r = r_ref[...]
    y0 = (jnp.sum(a0_ref[...], axis=0, keepdims=True) * d + z0_ref[...] * d2
          + r * bw0 + b20)
    y1 = (jnp.sum(a1_ref[...], axis=0, keepdims=True) * d + z1_ref[...] * d2
          + r * bw1 + b21)
    m = jnp.maximum(y0, y1)
    lse = m + jnp.log(jnp.exp(y0 - m) + jnp.exp(y1 - m))
    o0_ref[...] = y0 - lse
    o1_ref[...] = y1 - lse


def _final(a0, a1, dis, z0, z1, r, b1row, w2p, b2row):
    return pl.pallas_call(
        _final_body,
        out_shape=[jax.ShapeDtypeStruct((1, NN), _f32)] * 2,
    )(a0, a1, dis, z0, z1, r, b1row, w2p, b2row)


def kernel(x, edge_index, lstm_params, W1, b1, W2, b2):
    p = lstm_params
    bf0 = (p["bih_l0_d0"] + p["bhh_l0_d0"]).reshape(1, 4 * HID)
    bb0 = (p["bih_l0_d1"] + p["bhh_l0_d1"]).reshape(1, 4 * HID)
    xf0, xb0 = _proj0(x, p["Wih_l0_d0"].T, p["Wih_l0_d1"].T, bf0, bb0)
    h0f, h0b = _bilstm_scan(xf0, xb0, p["Whh_l0_d0"].T, p["Whh_l0_d1"].T)

    wf1 = p["Wih_l1_d0"].T
    wb1 = p["Wih_l1_d1"].T
    bf1 = (p["bih_l1_d0"] + p["bhh_l1_d0"]).reshape(1, 4 * HID)
    bb1 = (p["bih_l1_d1"] + p["bhh_l1_d1"]).reshape(1, 4 * HID)
    xf1, xb1 = _proj1(h0f, h0b, wf1[:HID], wf1[HID:], wb1[:HID], wb1[HID:],
                      bf1, bb1)
    h1f, h1b = _bilstm_scan(xf1, xb1, p["Whh_l1_d0"].T, p["Whh_l1_d1"].T)

    w2p = jnp.pad(W2, ((0, 0), (0, HID - W2.shape[1])))
    pmat = _pproj(h1f, h1b, W1[:HID], W1[HID:], w2p)
    p0 = pmat[:, 0]
    p1 = pmat[:, 1]

    src = edge_index[0]
    dst = edge_index[1]
    degp = _sc_degree(dst)
    dis_row = _norm(degp)
    dis = dis_row.reshape(NN)
    a0, a1, ar = _sc_aggregate(src, dst, [p0, p1], dis)
    z0, z1, r = _mid(a0, a1, ar, dis_row, p0.reshape(1, NN), p1.reshape(1, NN))
    c0, c1, _ = _sc_aggregate(src, dst, [z0.reshape(NN), z1.reshape(NN)], dis)
    b2row = jnp.pad(b2, (0, HID - b2.shape[0])).reshape(1, HID)
    o0, o1 = _final(c0, c1, dis_row, z0, z1, r, b1.reshape(1, 4 * HID), w2p,
                    b2row)
    return jnp.concatenate([o0, o1], axis=0).T


# speedup vs baseline: 12.1878x; 1.1355x over previous
import functools

import jax
import jax.numpy as jnp
from jax import lax
from jax.experimental import pallas as pl
from jax.experimental.pallas import tpu as pltpu
from jax.experimental.pallas import tpu_sc as plsc

NN = 10000
HID = 128
NE = 320000
NC, NS = 2, 16
NW = NC * NS
EPW = NE // NW
VL = 16
CHUNK = 1000
GRID = NN // CHUNK

_f32 = jnp.float32


def _proj0_body(x_ref, wf_ref, wb_ref, bf_ref, bb_ref, xf_ref, xb_ref):
    x = x_ref[...]
    xf_ref[...] = jnp.dot(x, wf_ref[...], preferred_element_type=_f32) + bf_ref[...]
    xb_ref[...] = jnp.dot(x, wb_ref[...], preferred_element_type=_f32) + bb_ref[...]


def _proj0(x, wfT, wbT, bf, bb):
    return pl.pallas_call(
        _proj0_body,
        grid=(GRID,),
        in_specs=[
            pl.BlockSpec((CHUNK, HID), lambda i: (i, 0)),
            pl.BlockSpec((HID, 4 * HID), lambda i: (0, 0)),
            pl.BlockSpec((HID, 4 * HID), lambda i: (0, 0)),
            pl.BlockSpec((1, 4 * HID), lambda i: (0, 0)),
            pl.BlockSpec((1, 4 * HID), lambda i: (0, 0)),
        ],
        out_specs=[
            pl.BlockSpec((CHUNK, 4 * HID), lambda i: (i, 0)),
            pl.BlockSpec((CHUNK, 4 * HID), lambda i: (i, 0)),
        ],
        out_shape=[
            jax.ShapeDtypeStruct((NN, 4 * HID), _f32),
            jax.ShapeDtypeStruct((NN, 4 * HID), _f32),
        ],
    )(x, wfT, wbT, bf, bb)


def _proj1_body(hf_ref, hb_ref, af_ref, bf_ref, ab_ref, bb_ref,
                cf_ref, cb_ref, xf_ref, xb_ref):
    hf = hf_ref[...]
    hb = hb_ref[...]
    xf_ref[...] = (jnp.dot(hf, af_ref[...], preferred_element_type=_f32)
                   + jnp.dot(hb, bf_ref[...], preferred_element_type=_f32)
                   + cf_ref[...])
    xb_ref[...] = (jnp.dot(hf, ab_ref[...], preferred_element_type=_f32)
                   + jnp.dot(hb, bb_ref[...], preferred_element_type=_f32)
                   + cb_ref[...])


def _proj1(hf, hb, afT, bfT, abT, bbT, cf, cb):
    return pl.pallas_call(
        _proj1_body,
        grid=(GRID,),
        in_specs=[
            pl.BlockSpec((CHUNK, HID), lambda i: (i, 0)),
            pl.BlockSpec((CHUNK, HID), lambda i: (i, 0)),
            pl.BlockSpec((HID, 4 * HID), lambda i: (0, 0)),
            pl.BlockSpec((HID, 4 * HID), lambda i: (0, 0)),
            pl.BlockSpec((HID, 4 * HID), lambda i: (0, 0)),
            pl.BlockSpec((HID, 4 * HID), lambda i: (0, 0)),
            pl.BlockSpec((1, 4 * HID), lambda i: (0, 0)),
            pl.BlockSpec((1, 4 * HID), lambda i: (0, 0)),
        ],
        out_specs=[
            pl.BlockSpec((CHUNK, 4 * HID), lambda i: (i, 0)),
            pl.BlockSpec((CHUNK, 4 * HID), lambda i: (i, 0)),
        ],
        out_shape=[
            jax.ShapeDtypeStruct((NN, 4 * HID), _f32),
            jax.ShapeDtypeStruct((NN, 4 * HID), _f32),
        ],
    )(hf, hb, afT, bfT, abT, bbT, cf, cb)


def _scan_body(xf_ref, xb_ref, wf_ref, wb_ref, hf_out, hb_out,
               hf_s, cf_s, hb_s, cb_s):
    @pl.when(pl.program_id(0) == 0)
    def _():
        hf_s[...] = jnp.zeros((1, HID), _f32)
        cf_s[...] = jnp.zeros((1, HID), _f32)
        hb_s[...] = jnp.zeros((1, HID), _f32)
        cb_s[...] = jnp.zeros((1, HID), _f32)

    wf = wf_ref[...]
    wb = wb_ref[...]

    def _sig(x):
        return 0.5 * jnp.tanh(0.5 * x) + 0.5

    def step(t, carry):
        hf, cf, hb, cb = carry
        gf = xf_ref[pl.ds(t, 1), :] + jnp.dot(hf, wf, preferred_element_type=_f32)
        i = _sig(gf[:, 0:HID])
        f = _sig(gf[:, HID:2 * HID])
        g = jnp.tanh(gf[:, 2 * HID:3 * HID])
        o = _sig(gf[:, 3 * HID:4 * HID])
        cf = f * cf + i * g
        hf = o * jnp.tanh(cf)
        hf_out[pl.ds(t, 1), :] = hf

        tb = CHUNK - 1 - t
        gb = xb_ref[pl.ds(tb, 1), :] + jnp.dot(hb, wb, preferred_element_type=_f32)
        ib = _sig(gb[:, 0:HID])
        fb = _sig(gb[:, HID:2 * HID])
        gg = jnp.tanh(gb[:, 2 * HID:3 * HID])
        ob = _sig(gb[:, 3 * HID:4 * HID])
        cb = fb * cb + ib * gg
        hb = ob * jnp.tanh(cb)
        hb_out[pl.ds(tb, 1), :] = hb
        return hf, cf, hb, cb

    carry = (hf_s[...], cf_s[...], hb_s[...], cb_s[...])
    hf, cf, hb, cb = lax.fori_loop(0, CHUNK, step, carry, unroll=2)
    hf_s[...] = hf
    cf_s[...] = cf
    hb_s[...] = hb
    cb_s[...] = cb


def _bilstm_scan(xf, xb, whfT, whbT):
    return pl.pallas_call(
        _scan_body,
        grid=(GRID,),
        in_specs=[
            pl.BlockSpec((CHUNK, 4 * HID), lambda i: (i, 0)),
            pl.BlockSpec((CHUNK, 4 * HID), lambda i: (GRID - 1 - i, 0)),
            pl.BlockSpec((HID, 4 * HID), lambda i: (0, 0)),
            pl.BlockSpec((HID, 4 * HID), lambda i: (0, 0)),
        ],
        out_specs=[
            pl.BlockSpec((CHUNK, HID), lambda i: (i, 0)),
            pl.BlockSpec((CHUNK, HID), lambda i: (GRID - 1 - i, 0)),
        ],
        out_shape=[
            jax.ShapeDtypeStruct((NN, HID), _f32),
            jax.ShapeDtypeStruct((NN, HID), _f32),
        ],
        scratch_shapes=[pltpu.VMEM((1, HID), _f32)] * 4,
    )(xf, xb, whfT, whbT)


def _pproj_body(hf_ref, hb_ref, w1a_ref, w1b_ref, w2p_ref, p_ref):
    wca = jnp.dot(w1a_ref[...], w2p_ref[...], preferred_element_type=_f32)
    wcb = jnp.dot(w1b_ref[...], w2p_ref[...], preferred_element_type=_f32)
    p_ref[...] = (jnp.dot(hf_ref[...], wca, preferred_element_type=_f32)
                  + jnp.dot(hb_ref[...], wcb, preferred_element_type=_f32))


def _pproj(hf, hb, w1a, w1b, w2p):
    return pl.pallas_call(
        _pproj_body,
        grid=(GRID,),
        in_specs=[
            pl.BlockSpec((CHUNK, HID), lambda i: (i, 0)),
            pl.BlockSpec((CHUNK, HID), lambda i: (i, 0)),
            pl.BlockSpec((HID, 4 * HID), lambda i: (0, 0)),
            pl.BlockSpec((HID, 4 * HID), lambda i: (0, 0)),
            pl.BlockSpec((4 * HID, HID), lambda i: (0, 0)),
        ],
        out_specs=pl.BlockSpec((CHUNK, HID), lambda i: (i, 0)),
        out_shape=jax.ShapeDtypeStruct((NN, HID), _f32),
    )(hf, hb, w1a, w1b, w2p)


def _sc_mesh():
    return plsc.VectorSubcoreMesh(core_axis_name="c", subcore_axis_name="s",
                                  num_cores=NC, num_subcores=NS)


def _zero_vmem(ref):
    def zstep(i, _):
        ref[pl.ds(i * VL, VL)] = jnp.zeros((VL,), _f32)
        return 0
    lax.fori_loop(0, NN // VL, zstep, 0)


def _deg_body(dst_hbm, out_hbm, idx_v, acc_v):
    wid = lax.axis_index("s") * NC + lax.axis_index("c")
    pltpu.sync_copy(dst_hbm.at[pl.ds(wid * EPW, EPW)], idx_v)
    _zero_vmem(acc_v)
    ones = jnp.ones((VL,), _f32)

    def step(i, _):
        d = idx_v[pl.ds(i * VL, VL)]
        plsc.addupdate_scatter(acc_v, [d], ones)
        return 0

    lax.fori_loop(0, EPW // VL, step, 0)
    pltpu.sync_copy(acc_v, out_hbm.at[wid])


def _sc_degree(dst):
    return pl.kernel(
        _deg_body,
        out_type=jax.ShapeDtypeStruct((NW, NN), _f32),
        mesh=_sc_mesh(),
        compiler_params=pltpu.CompilerParams(needs_layout_passes=False),
        scratch_types=[
            pltpu.VMEM((EPW,), jnp.int32),
            pltpu.VMEM((NN,), _f32),
        ],
    )(dst)


def _make_agg_body(nch):
    def body(src_hbm, dst_hbm, *rest):
        tabs_hbm = rest[:nch]
        dis_hbm = rest[nch]
        outs_hbm = rest[nch + 1:nch + 2 + nch]
        scr = rest[nch + 2 + nch:]
        src_v, dst_v = scr[0], scr[1]
        tab_v = scr[2:2 + nch]
        dis_v = scr[2 + nch]
        acc_v = scr[3 + nch:3 + nch + nch + 1]

        wid = lax.axis_index("s") * NC + lax.axis_index("c")
        pltpu.sync_copy(src_hbm.at[pl.ds(wid * EPW, EPW)], src_v)
        pltpu.sync_copy(dst_hbm.at[pl.ds(wid * EPW, EPW)], dst_v)
        for c in range(nch):
            pltpu.sync_copy(tabs_hbm[c], tab_v[c])
        pltpu.sync_copy(dis_hbm, dis_v)
        for a in acc_v:
            _zero_vmem(a)

        def step(i, _):
            s = src_v[pl.ds(i * VL, VL)]
            d = dst_v[pl.ds(i * VL, VL)]
            ds = plsc.load_gather(dis_v, [s])
            for c in range(nch):
                v = plsc.load_gather(tab_v[c], [s]) * ds
                plsc.addupdate_scatter(acc_v[c], [d], v)
            plsc.addupdate_scatter(acc_v[nch], [d], ds)
            return 0

        lax.fori_loop(0, EPW // VL, step, 0)
        for c in range(nch + 1):
            pltpu.sync_copy(acc_v[c], outs_hbm[c].at[wid])

    return body


def _sc_aggregate(src, dst, tabs, dis):
    nch = len(tabs)
    out_t = [jax.ShapeDtypeStruct((NW, NN), _f32)] * (nch + 1)
    scratch = ([pltpu.VMEM((EPW,), jnp.int32)] * 2
               + [pltpu.VMEM((NN,), _f32)] * (nch + 1)
               + [pltpu.VMEM((NN,), _f32)] * (nch + 1))
    return pl.kernel(
        _make_agg_body(nch),
        out_type=out_t,
        mesh=_sc_mesh(),
        compiler_params=pltpu.CompilerParams(needs_layout_passes=False),
        scratch_types=scratch,
    )(src, dst, *tabs, dis)


def _norm_body(degp_ref, dis_ref):
    deg = jnp.sum(degp_ref[...], axis=0, keepdims=True) + 1.0
    dis_ref[...] = lax.rsqrt(deg)


def _norm(degp):
    return pl.pallas_call(
        _norm_body,
        out_shape=jax.ShapeDtypeStruct((1, NN), _f32),
    )(degp)


def _mid_body(a0_ref, a1_ref, ar_ref, dis_ref, p0_ref, p1_ref,
              z0_ref, z1_ref, r_ref):
    d = dis_ref[...]
    d2 = d * d
    z0_ref[...] = jnp.sum(a0_ref[...], axis=0, keepdims=True) * d + p0_ref[...] * d2
    z1_ref[...] = jnp.sum(a1_ref[...], axis=0, keepdims=True) * d + p1_ref[...] * d2
    r_ref[...] = jnp.sum(ar_ref[...], axis=0, keepdims=True) * d + d2


def _mid(a0, a1, ar, dis, p0, p1):
    return pl.pallas_call(
        _mid_body,
        out_shape=[jax.ShapeDtypeStruct((1, NN), _f32)] * 3,
    )(a0, a1, ar, dis, p0, p1)


def _final_body(a0_ref, a1_ref, dis_ref, z0_ref, z1_ref, r_ref,
                b1_ref, w2p_ref, b2_ref, o0_ref, o1_ref):
    d = dis_ref[...]
    d2 = d * d
    bw = jnp.dot(b1_ref[...], w2p_ref[...], preferred_element_type=_f32)
    bw0 = bw[0:1, 0:1]
    bw1 = bw[0:1, 1:2]
    b20 = b2_ref[0:1, 0:1]
    b21 = b2_ref[0:1, 1:2]
    r = r_ref[...]
    y0 = (jnp.sum(a0_ref[...], axis=0, keepdims=True) * d + z0_ref[...] * d2
          + r * bw0 + b20)
    y1 = (jnp.sum(a1_ref[...], axis=0, keepdims=True) * d + z1_ref[...] * d2
          + r * bw1 + b21)
    m = jnp.maximum(y0, y1)
    lse = m + jnp.log(jnp.exp(y0 - m) + jnp.exp(y1 - m))
    o0_ref[...] = y0 - lse
    o1_ref[...] = y1 - lse


def _final(a0, a1, dis, z0, z1, r, b1row, w2p, b2row):
    return pl.pallas_call(
        _final_body,
        out_shape=[jax.ShapeDtypeStruct((1, NN), _f32)] * 2,
    )(a0, a1, dis, z0, z1, r, b1row, w2p, b2row)


def kernel(x, edge_index, lstm_params, W1, b1, W2, b2):
    p = lstm_params
    bf0 = (p["bih_l0_d0"] + p["bhh_l0_d0"]).reshape(1, 4 * HID)
    bb0 = (p["bih_l0_d1"] + p["bhh_l0_d1"]).reshape(1, 4 * HID)
    xf0, xb0 = _proj0(x, p["Wih_l0_d0"].T, p["Wih_l0_d1"].T, bf0, bb0)
    h0f, h0b = _bilstm_scan(xf0, xb0, p["Whh_l0_d0"].T, p["Whh_l0_d1"].T)

    wf1 = p["Wih_l1_d0"].T
    wb1 = p["Wih_l1_d1"].T
    bf1 = (p["bih_l1_d0"] + p["bhh_l1_d0"]).reshape(1, 4 * HID)
    bb1 = (p["bih_l1_d1"] + p["bhh_l1_d1"]).reshape(1, 4 * HID)
    xf1, xb1 = _proj1(h0f, h0b, wf1[:HID], wf1[HID:], wb1[:HID], wb1[HID:],
                      bf1, bb1)
    h1f, h1b = _bilstm_scan(xf1, xb1, p["Whh_l1_d0"].T, p["Whh_l1_d1"].T)

    w2p = jnp.pad(W2, ((0, 0), (0, HID - W2.shape[1])))
    pmat = _pproj(h1f, h1b, W1[:HID], W1[HID:], w2p)
    p0 = pmat[:, 0]
    p1 = pmat[:, 1]

    src = edge_index[0]
    dst = edge_index[1]
    degp = _sc_degree(dst)
    dis_row = _norm(degp)
    dis = dis_row.reshape(NN)
    a0, a1, ar = _sc_aggregate(src, dst, [p0, p1], dis)
    z0, z1, r = _mid(a0, a1, ar, dis_row, p0.reshape(1, NN), p1.reshape(1, NN))
    c0, c1, _ = _sc_aggregate(src, dst, [z0.reshape(NN), z1.reshape(NN)], dis)
    b2row = jnp.pad(b2, (0, HID - b2.shape[0])).reshape(1, HID)
    o0, o1 = _final(c0, c1, dis_row, z0, z1, r, b1.reshape(1, 4 * HID), w2p,
                    b2row)
    return jnp.concatenate([o0, o1], axis=0).T


# speedup vs baseline: 12.8802x; 1.0568x over previous
import functools

import jax
import jax.numpy as jnp
from jax import lax
from jax.experimental import pallas as pl
from jax.experimental.pallas import tpu as pltpu
from jax.experimental.pallas import tpu_sc as plsc

NN = 10000
HID = 128
NE = 320000
NC, NS = 2, 16
NW = NC * NS
EPW = NE // NW
VL = 16
CHUNK = 1000
GRID = NN // CHUNK

_f32 = jnp.float32


def _proj0_body(x_ref, wf_ref, wb_ref, bf_ref, bb_ref, xf_ref, xb_ref):
    x = x_ref[...]
    xf_ref[...] = jnp.dot(x, wf_ref[...], preferred_element_type=_f32) + bf_ref[...]
    xb_ref[...] = jnp.dot(x, wb_ref[...], preferred_element_type=_f32) + bb_ref[...]


def _proj0(x, wfT, wbT, bf, bb):
    return pl.pallas_call(
        _proj0_body,
        grid=(GRID,),
        in_specs=[
            pl.BlockSpec((CHUNK, HID), lambda i: (i, 0)),
            pl.BlockSpec((HID, 4 * HID), lambda i: (0, 0)),
            pl.BlockSpec((HID, 4 * HID), lambda i: (0, 0)),
            pl.BlockSpec((1, 4 * HID), lambda i: (0, 0)),
            pl.BlockSpec((1, 4 * HID), lambda i: (0, 0)),
        ],
        out_specs=[
            pl.BlockSpec((CHUNK, 4 * HID), lambda i: (i, 0)),
            pl.BlockSpec((CHUNK, 4 * HID), lambda i: (i, 0)),
        ],
        out_shape=[
            jax.ShapeDtypeStruct((NN, 4 * HID), _f32),
            jax.ShapeDtypeStruct((NN, 4 * HID), _f32),
        ],
    )(x, wfT, wbT, bf, bb)


def _proj1_body(hf_ref, hb_ref, af_ref, bf_ref, ab_ref, bb_ref,
                cf_ref, cb_ref, xf_ref, xb_ref):
    hf = hf_ref[...]
    hb = hb_ref[...]
    xf_ref[...] = (jnp.dot(hf, af_ref[...], preferred_element_type=_f32)
                   + jnp.dot(hb, bf_ref[...], preferred_element_type=_f32)
                   + cf_ref[...])
    xb_ref[...] = (jnp.dot(hf, ab_ref[...], preferred_element_type=_f32)
                   + jnp.dot(hb, bb_ref[...], preferred_element_type=_f32)
                   + cb_ref[...])


def _proj1(hf, hb, afT, bfT, abT, bbT, cf, cb):
    return pl.pallas_call(
        _proj1_body,
        grid=(GRID,),
        in_specs=[
            pl.BlockSpec((CHUNK, HID), lambda i: (i, 0)),
            pl.BlockSpec((CHUNK, HID), lambda i: (i, 0)),
            pl.BlockSpec((HID, 4 * HID), lambda i: (0, 0)),
            pl.BlockSpec((HID, 4 * HID), lambda i: (0, 0)),
            pl.BlockSpec((HID, 4 * HID), lambda i: (0, 0)),
            pl.BlockSpec((HID, 4 * HID), lambda i: (0, 0)),
            pl.BlockSpec((1, 4 * HID), lambda i: (0, 0)),
            pl.BlockSpec((1, 4 * HID), lambda i: (0, 0)),
        ],
        out_specs=[
            pl.BlockSpec((CHUNK, 4 * HID), lambda i: (i, 0)),
            pl.BlockSpec((CHUNK, 4 * HID), lambda i: (i, 0)),
        ],
        out_shape=[
            jax.ShapeDtypeStruct((NN, 4 * HID), _f32),
            jax.ShapeDtypeStruct((NN, 4 * HID), _f32),
        ],
    )(hf, hb, afT, bfT, abT, bbT, cf, cb)


def _scan_body(xf_ref, xb_ref, wf_ref, wb_ref, hf_out, hb_out,
               hf_s, cf_s, hb_s, cb_s):
    @pl.when(pl.program_id(0) == 0)
    def _():
        hf_s[...] = jnp.zeros((1, HID), _f32)
        cf_s[...] = jnp.zeros((1, HID), _f32)
        hb_s[...] = jnp.zeros((1, HID), _f32)
        cb_s[...] = jnp.zeros((1, HID), _f32)

    wf = wf_ref[...]
    wb = wb_ref[...]

    def _sig(x):
        return 0.5 * jnp.tanh(0.5 * x) + 0.5

    def step(t, carry):
        hf, cf, hb, cb = carry
        gf = xf_ref[pl.ds(t, 1), :] + jnp.dot(hf, wf, preferred_element_type=_f32)
        i = _sig(gf[:, 0:HID])
        f = _sig(gf[:, HID:2 * HID])
        g = jnp.tanh(gf[:, 2 * HID:3 * HID])
        o = _sig(gf[:, 3 * HID:4 * HID])
        cf = f * cf + i * g
        hf = o * jnp.tanh(cf)
        hf_out[pl.ds(t, 1), :] = hf

        tb = CHUNK - 1 - t
        gb = xb_ref[pl.ds(tb, 1), :] + jnp.dot(hb, wb, preferred_element_type=_f32)
        ib = _sig(gb[:, 0:HID])
        fb = _sig(gb[:, HID:2 * HID])
        gg = jnp.tanh(gb[:, 2 * HID:3 * HID])
        ob = _sig(gb[:, 3 * HID:4 * HID])
        cb = fb * cb + ib * gg
        hb = ob * jnp.tanh(cb)
        hb_out[pl.ds(tb, 1), :] = hb
        return hf, cf, hb, cb

    carry = (hf_s[...], cf_s[...], hb_s[...], cb_s[...])
    hf, cf, hb, cb = lax.fori_loop(0, CHUNK, step, carry, unroll=4)
    hf_s[...] = hf
    cf_s[...] = cf
    hb_s[...] = hb
    cb_s[...] = cb


def _bilstm_scan(xf, xb, whfT, whbT):
    return pl.pallas_call(
        _scan_body,
        grid=(GRID,),
        in_specs=[
            pl.BlockSpec((CHUNK, 4 * HID), lambda i: (i, 0)),
            pl.BlockSpec((CHUNK, 4 * HID), lambda i: (GRID - 1 - i, 0)),
            pl.BlockSpec((HID, 4 * HID), lambda i: (0, 0)),
            pl.BlockSpec((HID, 4 * HID), lambda i: (0, 0)),
        ],
        out_specs=[
            pl.BlockSpec((CHUNK, HID), lambda i: (i, 0)),
            pl.BlockSpec((CHUNK, HID), lambda i: (GRID - 1 - i, 0)),
        ],
        out_shape=[
            jax.ShapeDtypeStruct((NN, HID), _f32),
            jax.ShapeDtypeStruct((NN, HID), _f32),
        ],
        scratch_shapes=[pltpu.VMEM((1, HID), _f32)] * 4,
    )(xf, xb, whfT, whbT)


def _pproj_body(hf_ref, hb_ref, w1a_ref, w1b_ref, w2p_ref, p_ref):
    wca = jnp.dot(w1a_ref[...], w2p_ref[...], preferred_element_type=_f32)
    wcb = jnp.dot(w1b_ref[...], w2p_ref[...], preferred_element_type=_f32)
    p_ref[...] = (jnp.dot(hf_ref[...], wca, preferred_element_type=_f32)
                  + jnp.dot(hb_ref[...], wcb, preferred_element_type=_f32))


def _pproj(hf, hb, w1a, w1b, w2p):
    return pl.pallas_call(
        _pproj_body,
        grid=(GRID,),
        in_specs=[
            pl.BlockSpec((CHUNK, HID), lambda i: (i, 0)),
            pl.BlockSpec((CHUNK, HID), lambda i: (i, 0)),
            pl.BlockSpec((HID, 4 * HID), lambda i: (0, 0)),
            pl.BlockSpec((HID, 4 * HID), lambda i: (0, 0)),
            pl.BlockSpec((4 * HID, HID), lambda i: (0, 0)),
        ],
        out_specs=pl.BlockSpec((CHUNK, HID), lambda i: (i, 0)),
        out_shape=jax.ShapeDtypeStruct((NN, HID), _f32),
    )(hf, hb, w1a, w1b, w2p)


def _sc_mesh():
    return plsc.VectorSubcoreMesh(core_axis_name="c", subcore_axis_name="s",
                                  num_cores=NC, num_subcores=NS)


def _zero_vmem(ref):
    def zstep(i, _):
        ref[pl.ds(i * VL, VL)] = jnp.zeros((VL,), _f32)
        return 0
    lax.fori_loop(0, NN // VL, zstep, 0)


def _deg_body(dst_hbm, out_hbm, idx_v, acc_v):
    wid = lax.axis_index("s") * NC + lax.axis_index("c")
    pltpu.sync_copy(dst_hbm.at[pl.ds(wid * EPW, EPW)], idx_v)
    _zero_vmem(acc_v)
    ones = jnp.ones((VL,), _f32)

    def step(i, _):
        d = idx_v[pl.ds(i * VL, VL)]
        plsc.addupdate_scatter(acc_v, [d], ones)
        return 0

    lax.fori_loop(0, EPW // VL, step, 0)
    pltpu.sync_copy(acc_v, out_hbm.at[wid])


def _sc_degree(dst):
    return pl.kernel(
        _deg_body,
        out_type=jax.ShapeDtypeStruct((NW, NN), _f32),
        mesh=_sc_mesh(),
        compiler_params=pltpu.CompilerParams(needs_layout_passes=False),
        scratch_types=[
            pltpu.VMEM((EPW,), jnp.int32),
            pltpu.VMEM((NN,), _f32),
        ],
    )(dst)


def _make_agg_body(nch):
    def body(src_hbm, dst_hbm, *rest):
        tabs_hbm = rest[:nch]
        dis_hbm = rest[nch]
        outs_hbm = rest[nch + 1:nch + 2 + nch]
        scr = rest[nch + 2 + nch:]
        src_v, dst_v = scr[0], scr[1]
        tab_v = scr[2:2 + nch]
        dis_v = scr[2 + nch]
        acc_v = scr[3 + nch:3 + nch + nch + 1]

        wid = lax.axis_index("s") * NC + lax.axis_index("c")
        pltpu.sync_copy(src_hbm.at[pl.ds(wid * EPW, EPW)], src_v)
        pltpu.sync_copy(dst_hbm.at[pl.ds(wid * EPW, EPW)], dst_v)
        for c in range(nch):
            pltpu.sync_copy(tabs_hbm[c], tab_v[c])
        pltpu.sync_copy(dis_hbm, dis_v)
        for a in acc_v:
            _zero_vmem(a)

        def step(i, _):
            s = src_v[pl.ds(i * VL, VL)]
            d = dst_v[pl.ds(i * VL, VL)]
            ds = plsc.load_gather(dis_v, [s])
            for c in range(nch):
                v = plsc.load_gather(tab_v[c], [s]) * ds
                plsc.addupdate_scatter(acc_v[c], [d], v)
            plsc.addupdate_scatter(acc_v[nch], [d], ds)
            return 0

        lax.fori_loop(0, EPW // VL, step, 0)
        for c in range(nch + 1):
            pltpu.sync_copy(acc_v[c], outs_hbm[c].at[wid])

    return body


def _sc_aggregate(src, dst, tabs, dis):
    nch = len(tabs)
    out_t = [jax.ShapeDtypeStruct((NW, NN), _f32)] * (nch + 1)
    scratch = ([pltpu.VMEM((EPW,), jnp.int32)] * 2
               + [pltpu.VMEM((NN,), _f32)] * (nch + 1)
               + [pltpu.VMEM((NN,), _f32)] * (nch + 1))
    return pl.kernel(
        _make_agg_body(nch),
        out_type=out_t,
        mesh=_sc_mesh(),
        compiler_params=pltpu.CompilerParams(needs_layout_passes=False),
        scratch_types=scratch,
    )(src, dst, *tabs, dis)


def _norm_body(degp_ref, dis_ref):
    deg = jnp.sum(degp_ref[...], axis=0, keepdims=True) + 1.0
    dis_ref[...] = lax.rsqrt(deg)


def _norm(degp):
    return pl.pallas_call(
        _norm_body,
        out_shape=jax.ShapeDtypeStruct((1, NN), _f32),
    )(degp)


def _mid_body(a0_ref, a1_ref, ar_ref, dis_ref, p0_ref, p1_ref,
              z0_ref, z1_ref, r_ref):
    d = dis_ref[...]
    d2 = d * d
    z0_ref[...] = jnp.sum(a0_ref[...], axis=0, keepdims=True) * d + p0_ref[...] * d2
    z1_ref[...] = jnp.sum(a1_ref[...], axis=0, keepdims=True) * d + p1_ref[...] * d2
    r_ref[...] = jnp.sum(ar_ref[...], axis=0, keepdims=True) * d + d2


def _mid(a0, a1, ar, dis, p0, p1):
    return pl.pallas_call(
        _mid_body,
        out_shape=[jax.ShapeDtypeStruct((1, NN), _f32)] * 3,
    )(a0, a1, ar, dis, p0, p1)


def _final_body(a0_ref, a1_ref, dis_ref, z0_ref, z1_ref, r_ref,
                b1_ref, w2p_ref, b2_ref, o0_ref, o1_ref):
    d = dis_ref[...]
    d2 = d * d
    bw = jnp.dot(b1_ref[...], w2p_ref[...], preferred_element_type=_f32)
    bw0 = bw[0:1, 0:1]
    bw1 = bw[0:1, 1:2]
    b20 = b2_ref[0:1, 0:1]
    b21 = b2_ref[0:1, 1:2]
    r = r_ref[...]
    y0 = (jnp.sum(a0_ref[...], axis=0, keepdims=True) * d + z0_ref[...] * d2
          + r * bw0 + b20)
    y1 = (jnp.sum(a1_ref[...], axis=0, keepdims=True) * d + z1_ref[...] * d2
          + r * bw1 + b21)
    m = jnp.maximum(y0, y1)
    lse = m + jnp.log(jnp.exp(y0 - m) + jnp.exp(y1 - m))
    o0_ref[...] = y0 - lse
    o1_ref[...] = y1 - lse


def _final(a0, a1, dis, z0, z1, r, b1row, w2p, b2row):
    return pl.pallas_call(
        _final_body,
        out_shape=[jax.ShapeDtypeStruct((1, NN), _f32)] * 2,
    )(a0, a1, dis, z0, z1, r, b1row, w2p, b2row)


def kernel(x, edge_index, lstm_params, W1, b1, W2, b2):
    p = lstm_params
    bf0 = (p["bih_l0_d0"] + p["bhh_l0_d0"]).reshape(1, 4 * HID)
    bb0 = (p["bih_l0_d1"] + p["bhh_l0_d1"]).reshape(1, 4 * HID)
    xf0, xb0 = _proj0(x, p["Wih_l0_d0"].T, p["Wih_l0_d1"].T, bf0, bb0)
    h0f, h0b = _bilstm_scan(xf0, xb0, p["Whh_l0_d0"].T, p["Whh_l0_d1"].T)

    wf1 = p["Wih_l1_d0"].T
    wb1 = p["Wih_l1_d1"].T
    bf1 = (p["bih_l1_d0"] + p["bhh_l1_d0"]).reshape(1, 4 * HID)
    bb1 = (p["bih_l1_d1"] + p["bhh_l1_d1"]).reshape(1, 4 * HID)
    xf1, xb1 = _proj1(h0f, h0b, wf1[:HID], wf1[HID:], wb1[:HID], wb1[HID:],
                      bf1, bb1)
    h1f, h1b = _bilstm_scan(xf1, xb1, p["Whh_l1_d0"].T, p["Whh_l1_d1"].T)

    w2p = jnp.pad(W2, ((0, 0), (0, HID - W2.shape[1])))
    pmat = _pproj(h1f, h1b, W1[:HID], W1[HID:], w2p)
    p0 = pmat[:, 0]
    p1 = pmat[:, 1]

    src = edge_index[0]
    dst = edge_index[1]
    degp = _sc_degree(dst)
    dis_row = _norm(degp)
    dis = dis_row.reshape(NN)
    a0, a1, ar = _sc_aggregate(src, dst, [p0, p1], dis)
    z0, z1, r = _mid(a0, a1, ar, dis_row, p0.reshape(1, NN), p1.reshape(1, NN))
    c0, c1, _ = _sc_aggregate(src, dst, [z0.reshape(NN), z1.reshape(NN)], dis)
    b2row = jnp.pad(b2, (0, HID - b2.shape[0])).reshape(1, HID)
    o0, o1 = _final(c0, c1, dis_row, z0, z1, r, b1.reshape(1, 4 * HID), w2p,
                    b2row)
    return jnp.concatenate([o0, o1], axis=0).T
